# baseline (speedup 1.0000x reference)
import dataclasses
import functools

import jax
import jax.numpy as jnp
from jax import lax
from jax.experimental import pallas as pl
from jax.experimental.pallas import tpu as pltpu
from jax.experimental.pallas import tpu_sc as plsc

B = 4096
L = 200
LP = 256
EMBED = 64
VOCAB = 101
NC = 2
NS = 16
NW = NC * NS
RPW = B // NW
FLAT = B * L
NPAIR = FLAT // 2
CHUNK = 128
KPW = NPAIR // NW // CHUNK
PPW = NPAIR // NW

_CP = pltpu.CompilerParams()
if "needs_layout_passes" in pltpu.CompilerParams.__dataclass_fields__:
    _CP = dataclasses.replace(_CP, needs_layout_passes=False)


@functools.partial(
    pl.kernel,
    out_type=(
        jax.ShapeDtypeStruct((NPAIR, 2 * EMBED), jnp.float32),
        jax.ShapeDtypeStruct((B,), jnp.int32),
    ),
    mesh=plsc.VectorSubcoreMesh(core_axis_name="c", subcore_axis_name="s"),
    compiler_params=_CP,
    scratch_types=[
        pltpu.VMEM((B,), jnp.int32),
        pltpu.VMEM((B,), jnp.int32),
        pltpu.VMEM((256, 16), jnp.int32),
        pltpu.VMEM((256,), jnp.int32),
        pltpu.VMEM((256,), jnp.int32),
        pltpu.VMEM((16,), jnp.int32),
        pltpu.VMEM((16,), jnp.int32),
        pltpu.VMEM((B,), jnp.int32),
        pltpu.VMEM((B,), jnp.int32),
        pltpu.VMEM((RPW, LP), jnp.int32),
        pltpu.VMEM((KPW, CHUNK), jnp.int32),
        pltpu.VMEM((CHUNK, 2 * EMBED), jnp.float32),
        pltpu.VMEM((CHUNK, 2 * EMBED), jnp.float32),
        pltpu.SemaphoreType.DMA,
        pltpu.SemaphoreType.DMA,
        pltpu.SemaphoreType.DMA,
        pltpu.SemaphoreType.DMA,
        pltpu.SemaphoreType.DMA,
    ],
)
def _embed_pack_kernel(seq_hbm, lens_hbm, table_hbm, out_hbm, perm_hbm,
                       lens_v, rankp_v, run2_v, hist_v, offs_v, chunks_v,
                       sufch_v, perm_v, lsort_v, seq_v, gidx_v,
                       buf0, buf1, ssem,
                       gsem0, gsem1, wsem0, wsem1):
    wid = lax.axis_index("s") * NC + lax.axis_index("c")
    base = wid * RPW
    iota = lax.iota(jnp.int32, 16)
    zeros16 = jnp.zeros((16,), jnp.int32)
    bufs = (buf0, buf1)
    gsems = (gsem0, gsem1)
    wsems = (wsem0, wsem1)

    pltpu.sync_copy(lens_hbm, lens_v)

    @pl.loop(0, 256)
    def _zero_counts(t):
        run2_v[t] = zeros16

    @pl.loop(0, 16)
    def _zero_hist(t):
        hist_v[pl.ds(t * 16, 16)] = zeros16

    @pl.loop(0, 256)
    def _count(t):
        i_vec = iota * 256 + t
        lv = plsc.load_gather(lens_v, [i_vec])
        eb = plsc.load_gather(run2_v, [lv, iota])
        plsc.store_scatter(rankp_v, [i_vec], eb)
        plsc.store_scatter(run2_v, [lv, iota], eb + 1)

    @pl.loop(0, 201)
    def _prefix(v):
        tv = run2_v[v]
        cs = plsc.cumsum(tv)
        run2_v[v] = cs - tv
        vv = jnp.full((16,), v, jnp.int32)
        plsc.store_scatter(hist_v, [vv], cs, mask=iota == 15)

    @pl.loop(0, 16)
    def _chunk_sums(c):
        hv = hist_v[pl.ds(c * 16, 16)]
        s = jnp.sum(hv)
        plsc.store_scatter(chunks_v, [jnp.full((16,), c, jnp.int32)],
                           jnp.full((16,), s, jnp.int32), mask=iota == 0)

    ch = chunks_v[...]
    rch = lax.rev(ch, (0,))
    crch = plsc.cumsum(rch)
    sufch_v[...] = lax.rev(crch - rch, (0,))

    @pl.loop(0, 16)
    def _suffix(c):
        hv = hist_v[pl.ds(c * 16, 16)]
        rh = lax.rev(hv, (0,))
        crh = plsc.cumsum(rh)
        suf_in = lax.rev(crh - rh, (0,))
        bc = plsc.load_gather(sufch_v, [jnp.full((16,), c, jnp.int32)])
        offs_v[pl.ds(c * 16, 16)] = suf_in + bc

    @pl.loop(0, 256)
    def _place(t):
        i_vec = t * 16 + iota
        lv = lens_v[pl.ds(t * 16, 16)]
        strm = lax.shift_right_logical(i_vec, 8)
        pos = (plsc.load_gather(offs_v, [lv])
               + plsc.load_gather(run2_v, [lv, strm])
               + rankp_v[pl.ds(t * 16, 16)])
        plsc.store_scatter(perm_v, [pos], i_vec)
        plsc.store_scatter(lsort_v, [pos], lv)

    pltpu.sync_copy(perm_v.at[pl.ds(base, RPW)], perm_hbm.at[pl.ds(base, RPW)])

    pltpu.async_copy(seq_hbm.at[perm_v.at[pl.ds(base, RPW)]], seq_v,
                     ssem).wait()

    @pl.loop(0, KPW)
    def _mask(k):
        for v in range(CHUNK // 16):
            pe = k * CHUNK + v * 16 + iota

            def masked(f):
                r = lax.div(f, jnp.int32(L))
                p = lax.rem(f, jnp.int32(L))
                sv = plsc.load_gather(seq_v, [r, p])
                ln = plsc.load_gather(lsort_v, [base + r])
                return jnp.where(p < ln, sv, 0)

            a0 = masked(2 * pe)
            a1 = masked(2 * pe + 1)
            gidx_v[k, pl.ds(v * 16, 16)] = a0 * VOCAB + a1

    def out_slice(k):
        return out_hbm.at[pl.ds(wid * PPW + k * CHUNK, CHUNK)]

    def start_gather(k, b):
        pltpu.async_copy(table_hbm.at[gidx_v.at[k]], bufs[b], gsems[b])

    def wait_gather(k, b):
        pltpu.make_async_copy(table_hbm.at[gidx_v.at[k]], bufs[b],
                              gsems[b]).wait()

    def start_write(k, b):
        pltpu.async_copy(bufs[b], out_slice(k), wsems[b])

    def wait_write(k, b):
        pltpu.make_async_copy(bufs[b], out_slice(k), wsems[b]).wait()

    @pl.loop(0, KPW, step=2)
    def _main(k):
        for dk in range(2):
            kk = k + dk
            b = dk

            @pl.when(kk >= 2)
            def _drain():
                wait_write(kk - 2, b)

            start_gather(kk, b)
            wait_gather(kk, b)
            start_write(kk, b)

    wait_write(KPW - 2, 0)
    wait_write(KPW - 1, 1)


def kernel(seq_tensor, seq_lengths, embed_weight):
    seq = jnp.pad(seq_tensor.astype(jnp.int32), ((0, 0), (0, LP - L)))
    lens = seq_lengths.astype(jnp.int32)
    w = embed_weight.astype(jnp.float32)
    table2 = jnp.concatenate(
        [jnp.broadcast_to(w[:, None, :], (VOCAB, VOCAB, EMBED)),
         jnp.broadcast_to(w[None, :, :], (VOCAB, VOCAB, EMBED))],
        axis=-1).reshape(VOCAB * VOCAB, 2 * EMBED)
    flat, perm = _embed_pack_kernel(seq, lens, table2)
    return flat.reshape(B, L, EMBED), perm


# speedup vs baseline: 3.5776x; 3.5776x over previous
import dataclasses
import functools

import jax
import jax.numpy as jnp
from jax import lax
from jax.experimental import pallas as pl
from jax.experimental.pallas import tpu as pltpu
from jax.experimental.pallas import tpu_sc as plsc

B = 4096
L = 200
LP = 256
EMBED = 64
VOCAB = 101
NC = 2
NS = 16
NW = NC * NS
RPW = B // NW
FLAT = B * L
NPAIR = FLAT // 2
ERPC = 128
PRPC = ERPC // 2
KPW = FLAT // NW // ERPC
PPW = NPAIR // NW

_CP = pltpu.CompilerParams()
if "needs_layout_passes" in pltpu.CompilerParams.__dataclass_fields__:
    _CP = dataclasses.replace(_CP, needs_layout_passes=False)


@functools.partial(
    pl.kernel,
    out_type=(
        jax.ShapeDtypeStruct((FLAT * EMBED,), jnp.float32),
        jax.ShapeDtypeStruct((B,), jnp.int32),
    ),
    mesh=plsc.VectorSubcoreMesh(core_axis_name="c", subcore_axis_name="s"),
    compiler_params=_CP,
    scratch_types=[
        pltpu.VMEM((B,), jnp.int32),
        pltpu.VMEM((B,), jnp.int32),
        pltpu.VMEM((256, 16), jnp.int32),
        pltpu.VMEM((256,), jnp.int32),
        pltpu.VMEM((256,), jnp.int32),
        pltpu.VMEM((16,), jnp.int32),
        pltpu.VMEM((16,), jnp.int32),
        pltpu.VMEM((B,), jnp.int32),
        pltpu.VMEM((B,), jnp.int32),
        pltpu.VMEM((RPW, LP), jnp.int32),
        pltpu.VMEM((VOCAB * EMBED,), jnp.float32),
        pltpu.VMEM((ERPC * EMBED,), jnp.float32),
        pltpu.VMEM((ERPC * EMBED,), jnp.float32),
        pltpu.SemaphoreType.DMA,
        pltpu.SemaphoreType.DMA,
        pltpu.SemaphoreType.DMA,
    ],
)
def _embed_pack_kernel(seq_hbm, lens_hbm, table_hbm, out_hbm, perm_hbm,
                       lens_v, rankp_v, run2_v, hist_v, offs_v, chunks_v,
                       sufch_v, perm_v, lsort_v, seq_v, table_v,
                       buf0, buf1, ssem, wsem0, wsem1):
    wid = lax.axis_index("s") * NC + lax.axis_index("c")
    base = wid * RPW
    iota = lax.iota(jnp.int32, 16)
    zeros16 = jnp.zeros((16,), jnp.int32)
    bufs = (buf0, buf1)
    wsems = (wsem0, wsem1)

    pltpu.sync_copy(lens_hbm, lens_v)
    pltpu.sync_copy(table_hbm, table_v)

    @pl.loop(0, 256)
    def _zero_counts(t):
        run2_v[t] = zeros16

    @pl.loop(0, 16)
    def _zero_hist(t):
        hist_v[pl.ds(t * 16, 16)] = zeros16

    @pl.loop(0, 256)
    def _count(t):
        i_vec = iota * 256 + t
        lv = plsc.load_gather(lens_v, [i_vec])
        eb = plsc.load_gather(run2_v, [lv, iota])
        plsc.store_scatter(rankp_v, [i_vec], eb)
        plsc.store_scatter(run2_v, [lv, iota], eb + 1)

    @pl.loop(0, 201)
    def _prefix(v):
        tv = run2_v[v]
        cs = plsc.cumsum(tv)
        run2_v[v] = cs - tv
        vv = jnp.full((16,), v, jnp.int32)
        plsc.store_scatter(hist_v, [vv], cs, mask=iota == 15)

    @pl.loop(0, 16)
    def _chunk_sums(c):
        hv = hist_v[pl.ds(c * 16, 16)]
        s = jnp.sum(hv)
        plsc.store_scatter(chunks_v, [jnp.full((16,), c, jnp.int32)],
                           jnp.full((16,), s, jnp.int32), mask=iota == 0)

    ch = chunks_v[...]
    rch = lax.rev(ch, (0,))
    crch = plsc.cumsum(rch)
    sufch_v[...] = lax.rev(crch - rch, (0,))

    @pl.loop(0, 16)
    def _suffix(c):
        hv = hist_v[pl.ds(c * 16, 16)]
        rh = lax.rev(hv, (0,))
        crh = plsc.cumsum(rh)
        suf_in = lax.rev(crh - rh, (0,))
        bc = plsc.load_gather(sufch_v, [jnp.full((16,), c, jnp.int32)])
        offs_v[pl.ds(c * 16, 16)] = suf_in + bc

    @pl.loop(0, 256)
    def _place(t):
        i_vec = t * 16 + iota
        lv = lens_v[pl.ds(t * 16, 16)]
        strm = lax.shift_right_logical(i_vec, 8)
        pos = (plsc.load_gather(offs_v, [lv])
               + plsc.load_gather(run2_v, [lv, strm])
               + rankp_v[pl.ds(t * 16, 16)])
        plsc.store_scatter(perm_v, [pos], i_vec)
        plsc.store_scatter(lsort_v, [pos], lv)

    pltpu.sync_copy(perm_v.at[pl.ds(base, RPW)], perm_hbm.at[pl.ds(base, RPW)])

    pltpu.async_copy(seq_hbm.at[perm_v.at[pl.ds(base, RPW)]], seq_v,
                     ssem).wait()

    def out_slice(k):
        return out_hbm.at[pl.ds(wid * KPW * ERPC * EMBED + k * ERPC * EMBED,
                                ERPC * EMBED)]

    @pl.loop(0, KPW, step=2)
    def _build(k):
        for dk in range(2):
            kk = k + dk
            buf = bufs[dk]
            wsem = wsems[dk]

            @pl.when(kk >= 2)
            def _drain():
                pltpu.make_async_copy(buf, out_slice(kk - 2), wsem).wait()

            for g in range(ERPC // 16):
                f = kk * ERPC + g * 16 + iota
                r = lax.div(f, jnp.int32(L))
                p = lax.rem(f, jnp.int32(L))
                sv = plsc.load_gather(seq_v, [r, p])
                ln = plsc.load_gather(lsort_v, [base + r])
                ids = jnp.where(p < ln, sv, 0)
                src0 = ids * EMBED
                dst0 = (g * 16 + iota) * EMBED
                for c in range(EMBED):
                    vals = plsc.load_gather(table_v, [src0 + c])
                    plsc.store_scatter(buf, [dst0 + c], vals)

            pltpu.async_copy(buf, out_slice(kk), wsem)

    pltpu.make_async_copy(buf0, out_slice(KPW - 2), wsem0).wait()
    pltpu.make_async_copy(buf1, out_slice(KPW - 1), wsem1).wait()


def kernel(seq_tensor, seq_lengths, embed_weight):
    seq = jnp.pad(seq_tensor.astype(jnp.int32), ((0, 0), (0, LP - L)))
    lens = seq_lengths.astype(jnp.int32)
    w = embed_weight.astype(jnp.float32).reshape(VOCAB * EMBED)
    flat, perm = _embed_pack_kernel(seq, lens, w)
    return flat.reshape(B, L, EMBED), perm
